# baseline (speedup 1.0000x reference)
import numpy as np

import jax
import jax.numpy as jnp
from jax import lax
from jax.experimental import pallas as pl
from jax.experimental.pallas import tpu as pltpu

SH = 512
BMA = 1000


def _proj_body(x_ref, w1_ref, b1_ref, w2_ref, s0_ref):
    h = lax.dot_general(x_ref[...], w1_ref[...], (((1,), (1,)), ((), ())),
                        preferred_element_type=jnp.float32)
    h = jnp.maximum(h + b1_ref[...], 0.0)
    s0_ref[...] = lax.dot_general(h, w2_ref[...], (((1,), (1,)), ((), ())),
                                  preferred_element_type=jnp.float32)


def _dot(a, b):
    return lax.dot_general(a, b, (((1,), (0,)), ((), ())),
                           precision=lax.Precision.DEFAULT,
                           preferred_element_type=jnp.float32)


def _make_sweep1_body(n, ns, hlast):
    last = ns - 1

    def dma_op(idx, rr_ref, l_ref, buf, sem, do_start):
        r = rr_ref[idx]
        roff = pl.multiple_of(r * SH, SH)
        for h, cond in ((SH, r < last), (hlast, r == last)):
            @pl.when(cond)
            def _(h=h):
                cp = pltpu.make_async_copy(
                    l_ref.at[pl.ds(roff, h), :],
                    buf.at[pl.ds(0, h), :], sem)
                if do_start:
                    cp.start()
                else:
                    cp.wait()

    def body(rr_ref, l_ref, s0_ref, s1_ref, u1_ref, buf0, buf1, sem0, sem1):
        t = pl.program_id(0)

        @pl.when(t == 0)
        def _():
            s1_ref[...] = jnp.zeros(s1_ref.shape, jnp.float32)
            dma_op(0, rr_ref, l_ref, buf0, sem0, True)

        @pl.when(jnp.logical_and(t + 1 < ns, t % 2 == 0))
        def _():
            dma_op(t + 1, rr_ref, l_ref, buf1, sem1, True)

        @pl.when(jnp.logical_and(t + 1 < ns, t % 2 == 1))
        def _():
            dma_op(t + 1, rr_ref, l_ref, buf0, sem0, True)

        def compute(buf, sem):
            r = rr_ref[t]
            dma_op(t, rr_ref, l_ref, buf, sem, False)
            roff = pl.multiple_of(r * SH, SH)
            for h, cond in ((SH, r < last), (hlast, r == last)):
                @pl.when(cond)
                def _(h=h):
                    panel = buf[...]
                    d1 = _dot(panel, s0_ref[...])
                    s1_ref[pl.ds(roff, h), :] = d1[:h, :]
                    d2 = _dot(panel, s1_ref[...])
                    u1_ref[pl.ds(roff, h), :] = d2[:h, :]

        @pl.when(t % 2 == 0)
        def _():
            compute(buf0, sem0)

        @pl.when(t % 2 == 1)
        def _():
            compute(buf1, sem1)

    return body


def _sweep2_body(pr_ref, pc_ref, ff_ref, fd_ref, fl_ref, l_ref, s1f_ref,
                 s0_ref, s1b_ref, u1_ref, th_ref, b2_ref, out_ref):
    t = pl.program_id(0)
    c = pc_ref[t]

    @pl.when(ff_ref[t] == 1)
    def _():
        out_ref[...] = u1_ref[...]

    @pl.when(fd_ref[t] == 1)
    def _():
        cof = pl.multiple_of(c * SH, SH)
        out_ref[...] += _dot(l_ref[...], s1f_ref[pl.ds(cof, SH), :])

    @pl.when(fl_ref[t] == 1)
    def _():
        u = out_ref[...]
        y = (th_ref[0:1, :] * s0_ref[...] + th_ref[1:2, :] * s1b_ref[...]
             + 2.0 * th_ref[2:3, :] * u + b2_ref[...])
        m = jnp.max(y, axis=1, keepdims=True)
        lse = jnp.log(jnp.sum(jnp.exp(y - m), axis=1, keepdims=True)) + m
        out_ref[...] = y - lse


def _sweep2_schedule(ns):
    steps = []

    def stripe(r, cols):
        if not cols:
            steps.append((r, 0, 1, 0, 1))
            return
        for j, c in enumerate(cols):
            steps.append((r, c, 1 if j == 0 else 0, 1,
                          1 if j == len(cols) - 1 else 0))

    stripe(ns - 1, list(range(ns - 1)))
    for r in range(ns - 1):
        stripe(r, list(range(r + 1, ns - 1)))
    arr = np.asarray(steps, np.int32)
    return arr[:, 0], arr[:, 1], arr[:, 2], arr[:, 3], arr[:, 4]


def kernel(x, L, W1, b1, W2, b2, thetas):
    N, F = x.shape
    H = W1.shape[0]
    C = W2.shape[0]
    ns = -(-N // SH)
    hlast = N - (ns - 1) * SH

    s0 = pl.pallas_call(
        _proj_body,
        grid=(N // BMA,),
        in_specs=[
            pl.BlockSpec((BMA, F), lambda i: (i, 0)),
            pl.BlockSpec((H, F), lambda i: (0, 0)),
            pl.BlockSpec((1, H), lambda i: (0, 0)),
            pl.BlockSpec((C, H), lambda i: (0, 0)),
        ],
        out_specs=pl.BlockSpec((BMA, C), lambda i: (i, 0)),
        out_shape=jax.ShapeDtypeStruct((N, C), jnp.float32),
    )(x, W1, b1.reshape(1, H), W2)

    rr = np.asarray([ns - 1] + list(range(ns - 1)), np.int32)

    s1, u1 = pl.pallas_call(
        _make_sweep1_body(N, ns, hlast),
        grid_spec=pltpu.PrefetchScalarGridSpec(
            num_scalar_prefetch=1,
            grid=(ns,),
            in_specs=[
                pl.BlockSpec(memory_space=pltpu.MemorySpace.HBM),
                pl.BlockSpec((N, C), lambda i, *_: (0, 0)),
            ],
            out_specs=[
                pl.BlockSpec((N, C), lambda i, *_: (0, 0)),
                pl.BlockSpec((N, C), lambda i, *_: (0, 0)),
            ],
            scratch_shapes=[
                pltpu.VMEM((SH, N), jnp.float32),
                pltpu.VMEM((SH, N), jnp.float32),
                pltpu.SemaphoreType.DMA,
                pltpu.SemaphoreType.DMA,
            ],
        ),
        out_shape=[jax.ShapeDtypeStruct((N, C), jnp.float32),
                   jax.ShapeDtypeStruct((N, C), jnp.float32)],
        compiler_params=pltpu.CompilerParams(
            dimension_semantics=("arbitrary",)),
    )(jnp.asarray(rr), L, s0)

    pr, pc, ff, fd, fl = _sweep2_schedule(ns)
    nsteps = pr.shape[0]

    th = jnp.broadcast_to(
        jnp.stack([thetas[0] - thetas[3], thetas[1] + thetas[2],
                   thetas[3]]).reshape(-1, 1), (3, C))

    out = pl.pallas_call(
        _sweep2_body,
        grid_spec=pltpu.PrefetchScalarGridSpec(
            num_scalar_prefetch=5,
            grid=(nsteps,),
            in_specs=[
                pl.BlockSpec((SH, SH), lambda i, pr, pc, *_: (pr[i], pc[i])),
                pl.BlockSpec((N, C), lambda i, *_: (0, 0)),
                pl.BlockSpec((SH, C), lambda i, pr, *_: (pr[i], 0)),
                pl.BlockSpec((SH, C), lambda i, pr, *_: (pr[i], 0)),
                pl.BlockSpec((SH, C), lambda i, pr, *_: (pr[i], 0)),
                pl.BlockSpec((3, C), lambda i, *_: (0, 0)),
                pl.BlockSpec((1, C), lambda i, *_: (0, 0)),
            ],
            out_specs=pl.BlockSpec((SH, C), lambda i, pr, *_: (pr[i], 0)),
        ),
        out_shape=jax.ShapeDtypeStruct((N, C), jnp.float32),
        compiler_params=pltpu.CompilerParams(
            dimension_semantics=("arbitrary",)),
    )(jnp.asarray(pr), jnp.asarray(pc), jnp.asarray(ff), jnp.asarray(fd),
      jnp.asarray(fl), L, s1, s0, s1, u1, th, b2.reshape(1, C))

    return out


# speedup vs baseline: 1.4991x; 1.4991x over previous
import jax
import jax.numpy as jnp
from jax import lax
from jax.experimental import pallas as pl
from jax.experimental.pallas import tpu as pltpu

BM = 400
BMA = 1000


def _proj_body(x_ref, w1_ref, b1_ref, w2_ref, s0_ref):
    h = lax.dot_general(x_ref[...], w1_ref[...], (((1,), (1,)), ((), ())),
                        preferred_element_type=jnp.float32)
    h = jnp.maximum(h + b1_ref[...], 0.0)
    s0_ref[...] = lax.dot_general(h, w2_ref[...], (((1,), (1,)), ((), ())),
                                  preferred_element_type=jnp.float32)


def _pass1_body(l_ref, s0_ref, s1_ref):
    s1_ref[...] = lax.dot_general(
        l_ref[...], s0_ref[...], (((1,), (0,)), ((), ())),
        precision=lax.Precision.DEFAULT,
        preferred_element_type=jnp.float32)


def _pass2_body(l_ref, s1in_ref, s0_ref, s1_ref, th_ref, b2_ref, out_ref):
    u = lax.dot_general(
        l_ref[...], s1in_ref[...], (((1,), (0,)), ((), ())),
        precision=lax.Precision.DEFAULT,
        preferred_element_type=jnp.float32)
    y = (th_ref[0:1, :] * s0_ref[...] + th_ref[1:2, :] * s1_ref[...]
         + 2.0 * th_ref[2:3, :] * u + b2_ref[...])
    m = jnp.max(y, axis=1, keepdims=True)
    lse = jnp.log(jnp.sum(jnp.exp(y - m), axis=1, keepdims=True)) + m
    out_ref[...] = y - lse


def kernel(x, L, W1, b1, W2, b2, thetas):
    N, F = x.shape
    H = W1.shape[0]
    C = W2.shape[0]
    ni = N // BM

    s0 = pl.pallas_call(
        _proj_body,
        grid=(N // BMA,),
        in_specs=[
            pl.BlockSpec((BMA, F), lambda i: (i, 0)),
            pl.BlockSpec((H, F), lambda i: (0, 0)),
            pl.BlockSpec((1, H), lambda i: (0, 0)),
            pl.BlockSpec((C, H), lambda i: (0, 0)),
        ],
        out_specs=pl.BlockSpec((BMA, C), lambda i: (i, 0)),
        out_shape=jax.ShapeDtypeStruct((N, C), jnp.float32),
    )(x, W1, b1.reshape(1, H), W2)

    s1 = pl.pallas_call(
        _pass1_body,
        grid=(ni,),
        in_specs=[
            pl.BlockSpec((BM, N), lambda i: (i, 0)),
            pl.BlockSpec((N, C), lambda i: (0, 0)),
        ],
        out_specs=pl.BlockSpec((BM, C), lambda i: (i, 0)),
        out_shape=jax.ShapeDtypeStruct((N, C), jnp.float32),
        compiler_params=pltpu.CompilerParams(
            dimension_semantics=("parallel",)),
    )(L, s0)

    th = jnp.broadcast_to(
        jnp.stack([thetas[0] - thetas[3], thetas[1] + thetas[2],
                   thetas[3]]).reshape(-1, 1), (3, C))
    out = pl.pallas_call(
        _pass2_body,
        grid=(ni,),
        in_specs=[
            pl.BlockSpec((BM, N), lambda i: (i, 0)),
            pl.BlockSpec((N, C), lambda i: (0, 0)),
            pl.BlockSpec((BM, C), lambda i: (i, 0)),
            pl.BlockSpec((BM, C), lambda i: (i, 0)),
            pl.BlockSpec((3, C), lambda i: (0, 0)),
            pl.BlockSpec((1, C), lambda i: (0, 0)),
        ],
        out_specs=pl.BlockSpec((BM, C), lambda i: (i, 0)),
        out_shape=jax.ShapeDtypeStruct((N, C), jnp.float32),
        compiler_params=pltpu.CompilerParams(
            dimension_semantics=("parallel",)),
    )(L, s1, s0, s1, th, b2.reshape(1, C))

    return out
